# baseline (speedup 1.0000x reference)
import functools

import jax
import jax.numpy as jnp
from jax import lax
from jax.experimental import pallas as pl
from jax.experimental.pallas import tpu as pltpu
from jax.experimental.pallas import tpu_sc as plsc


@functools.lru_cache(maxsize=None)
def _make_sc_gather(N, V, D, C, NBUF):
    info = plsc.get_sparse_core_info()
    NC, NS = info.num_cores, info.num_subcores
    NW = NC * NS
    assert N % NW == 0
    n_per_w = N // NW
    assert n_per_w % C == 0 and C % 8 == 0
    n_chunks = n_per_w // C
    assert n_chunks % NBUF == 0 and n_chunks >= NBUF >= 2
    mesh = plsc.VectorSubcoreMesh(core_axis_name="c", subcore_axis_name="s")

    @functools.partial(
        pl.kernel,
        mesh=mesh,
        compiler_params=pltpu.CompilerParams(use_tc_tiling_on_sc=False),
        out_type=jax.ShapeDtypeStruct((N, D), jnp.float32),
        scratch_types=(
            [pltpu.VMEM((n_per_w,), jnp.int32)]
            + [pltpu.VMEM((C, D), jnp.float32) for _ in range(NBUF)]
            + [pltpu.SemaphoreType.DMA for _ in range(2 * NBUF)]
        ),
    )
    def gather_kernel(table_hbm, idx_hbm, out_hbm, idx_v, *rest):
        bufs_only = rest[:NBUF]
        gsems = rest[NBUF:2 * NBUF]
        ssems = rest[2 * NBUF:3 * NBUF]
        wid = lax.axis_index("s") * NC + lax.axis_index("c")
        base = wid * n_per_w
        pltpu.sync_copy(idx_hbm.at[pl.ds(base, n_per_w)], idx_v)

        def start_gather(i, s):
            pltpu.async_copy(
                table_hbm.at[idx_v.at[pl.ds(i * C, C)]], bufs_only[s], gsems[s])

        def wait_gather(s):
            pltpu.make_async_copy(
                table_hbm.at[idx_v.at[pl.ds(0, C)]], bufs_only[s],
                gsems[s]).wait()

        def start_scatter(i, s):
            pltpu.async_copy(
                bufs_only[s], out_hbm.at[pl.ds(base + i * C, C)], ssems[s])

        def wait_scatter(s):
            pltpu.make_async_copy(
                bufs_only[s], out_hbm.at[pl.ds(base, C)], ssems[s]).wait()

        for j in range(NBUF - 1):
            start_gather(j, j)

        def group_body(g, carry):
            for b in range(NBUF):
                i = g * NBUF + b
                pb = (b - 1) % NBUF

                @pl.when(i + NBUF - 1 < n_chunks)
                def _():
                    @pl.when(i >= 1)
                    def _():
                        wait_scatter(pb)

                    start_gather(i + NBUF - 1, pb)

                wait_gather(b)
                start_scatter(i, b)
            return carry

        lax.fori_loop(0, n_chunks // NBUF, group_body, 0)
        for s in range(NBUF):
            wait_scatter(s)

    return gather_kernel


def kernel(contexts, table):
    B, S = contexts.shape
    V, D = table.shape
    N = B * S
    idx = contexts.reshape(N).astype(jnp.int32)
    out = _make_sc_gather(N, V, D, 16, 4)(table, idx)
    return out.reshape(B, S, D)
